# baseline (speedup 1.0000x reference)
import functools

import jax
import jax.numpy as jnp
from jax import lax
from jax.experimental import pallas as pl
from jax.experimental.pallas import tpu as pltpu
from jax.experimental.pallas import tpu_sc as plsc

S = 2048
DIM = 768
E = 64
ED = 128
BLOCK = 32
G = S // BLOCK + E
P = G * BLOCK
TB = 256

NC, NS = 2, 16
NW = NC * NS
CHUNK = S // NS
ZCHUNK = P // NS
PW = P // NW
HPW = PW // 2
UCHUNK = S // NW


def _gate_kernel(x_ref, gw_ref, bias_ref, idx_ref, wgt_ref):
    logits = jax.lax.dot_general(
        x_ref[...], gw_ref[...], (((1,), (1,)), ((), ())),
        preferred_element_type=jnp.float32,
    )
    scores = jax.nn.sigmoid(logits + bias_ref[...])
    m = jnp.max(scores, axis=1, keepdims=True)
    eids = jax.lax.broadcasted_iota(jnp.int32, scores.shape, 1)
    idx_ref[...] = jnp.min(jnp.where(scores == m, eids, E), axis=1, keepdims=True)
    wgt_ref[...] = m / (m + 1e-6)


NCH = 8
CROWS = P // NCH
CBLK = CROWS // BLOCK


def _ffn_kernel(be_ref, seg_ref, xs_hbm, w1_hbm, w2_hbm, ws_hbm, y_hbm,
                xsv, yv, wsv, w1b, w2b, xsems, wsem, w3sems, osem):
    nseg = seg_ref[E]
    for ch in range(NCH):
        pltpu.make_async_copy(xs_hbm.at[pl.ds(ch * CROWS, CROWS)],
                              xsv.at[pl.ds(ch * CROWS, CROWS)],
                              xsems.at[ch]).start()
    pltpu.make_async_copy(ws_hbm, wsv, wsem).start()
    e0 = seg_ref[0]
    pltpu.make_async_copy(w1_hbm.at[e0], w1b.at[0], w3sems.at[0, 0]).start()
    pltpu.make_async_copy(w2_hbm.at[e0], w2b.at[0], w3sems.at[0, 1]).start()

    @pl.when(nseg > 1)
    def _():
        e1 = seg_ref[1]
        pltpu.make_async_copy(w1_hbm.at[e1], w1b.at[1], w3sems.at[1, 0]).start()
        pltpu.make_async_copy(w2_hbm.at[e1], w2b.at[1], w3sems.at[1, 1]).start()

    pltpu.make_async_copy(ws_hbm, wsv, wsem).wait()

    def gloop(g, carry):
        k, s4, hprev = carry
        gp = jnp.maximum(g - 1, 0)
        y = jnp.dot(hprev, w2b[s4], preferred_element_type=jnp.float32)
        yv[pl.ds(gp * BLOCK, BLOCK), :] = y * wsv[pl.ds(gp * BLOCK, BLOCK), :]

        e = be_ref[g]
        prv = be_ref[gp]
        is_new = jnp.logical_or(g == 0, e != prv)
        k = jnp.where(is_new, k + 1, k)
        s4 = jnp.where(is_new, (s4 + 1) & 3, s4)

        @pl.when((g & (CBLK - 1)) == 0)
        def _():
            ch = g >> 4
            pltpu.make_async_copy(xs_hbm.at[pl.ds(ch * CROWS, CROWS)],
                                  xsv.at[pl.ds(ch * CROWS, CROWS)],
                                  xsems.at[ch]).wait()

            @pl.when(g > 0)
            def _():
                pltpu.make_async_copy(
                    yv.at[pl.ds((ch - 1) * CROWS, CROWS)],
                    y_hbm.at[pl.ds((ch - 1) * CROWS, CROWS)],
                    osem).start()

        @pl.when(jnp.logical_and(is_new, k < nseg))
        def _():
            pltpu.make_async_copy(w1_hbm.at[e], w1b.at[s4],
                                  w3sems.at[s4, 0]).wait()
            pltpu.make_async_copy(w2_hbm.at[e], w2b.at[s4],
                                  w3sems.at[s4, 1]).wait()
            nk = k + 2

            @pl.when(nk < nseg)
            def _():
                tgt = (s4 + 2) & 3
                ne = seg_ref[nk]
                pltpu.make_async_copy(w1_hbm.at[ne], w1b.at[tgt],
                                      w3sems.at[tgt, 0]).start()
                pltpu.make_async_copy(w2_hbm.at[ne], w2b.at[tgt],
                                      w3sems.at[tgt, 1]).start()

        xblk = xsv[pl.ds(g * BLOCK, BLOCK), :]
        h = jnp.dot(xblk, w1b[s4], preferred_element_type=jnp.float32)
        h = h * jax.nn.sigmoid(h)
        return (k, s4, h)

    k, s4, hlast = lax.fori_loop(
        0, G, gloop,
        (jnp.int32(-1), jnp.int32(3), jnp.zeros((BLOCK, ED), jnp.float32)))
    yl = jnp.dot(hlast, w2b[s4], preferred_element_type=jnp.float32)
    yv[pl.ds((G - 1) * BLOCK, BLOCK), :] = (
        yl * wsv[pl.ds((G - 1) * BLOCK, BLOCK), :])
    pltpu.make_async_copy(yv.at[pl.ds((NCH - 1) * CROWS, CROWS)],
                          y_hbm.at[pl.ds((NCH - 1) * CROWS, CROWS)],
                          osem).start()
    for ch in range(NCH):
        pltpu.make_async_copy(yv.at[pl.ds(ch * CROWS, CROWS)],
                              y_hbm.at[pl.ds(ch * CROWS, CROWS)],
                              osem).wait()


_sc_mesh = plsc.VectorSubcoreMesh(core_axis_name="c", subcore_axis_name="s")


@functools.partial(
    pl.kernel,
    out_type=[
        jax.ShapeDtypeStruct((P, DIM), jnp.float32),
        jax.ShapeDtypeStruct((P,), jnp.float32),
        jax.ShapeDtypeStruct((G,), jnp.int32),
        jax.ShapeDtypeStruct((S,), jnp.int32),
        jax.ShapeDtypeStruct((E + 16,), jnp.int32),
    ],
    mesh=_sc_mesh,
    scratch_types=[
        pltpu.VMEM((CHUNK,), jnp.int32),
        pltpu.VMEM((CHUNK,), jnp.float32),
        pltpu.VMEM((NS * E,), jnp.int32),
        pltpu.VMEM((E,), jnp.int32),
        pltpu.VMEM((E,), jnp.int32),
        pltpu.VMEM((E,), jnp.int32),
        pltpu.VMEM((E,), jnp.int32),
        pltpu.VMEM((E,), jnp.int32),
        pltpu.VMEM((E + 16,), jnp.int32),
        pltpu.VMEM((CHUNK,), jnp.int32),
        pltpu.VMEM((CHUNK,), jnp.int32),
        pltpu.VMEM((G,), jnp.int32),
        pltpu.VMEM((ZCHUNK,), jnp.int32),
        pltpu.VMEM((ZCHUNK,), jnp.float32),
        pltpu.VMEM((HPW,), jnp.int32),
        pltpu.VMEM((HPW, DIM), jnp.float32),
        pltpu.VMEM_SHARED((NS * E,), jnp.int32),
        pltpu.VMEM_SHARED((P,), jnp.int32),
        pltpu.VMEM_SHARED((P,), jnp.float32),
        pltpu.SemaphoreType.DMA,
    ],
    compiler_params=pltpu.CompilerParams(needs_layout_passes=False),
)
def _sc_route(top1_hbm, wgt_hbm, x_hbm, xs_hbm, ws_hbm, be_hbm, pos_hbm,
              seg_hbm, idx_v, wgt_v, allhist_v, hist_v, rb_v, po_v, runh_v,
              ends_v, seglist_v, plist_v, tok_v, be_v, zi_v, zf_v,
              gid_v, rows_v, hist_sh, ids_sh, ws_sh, sem):
    c = lax.axis_index("c")
    s = lax.axis_index("s")
    base = s * CHUNK
    iota = lax.broadcasted_iota(jnp.int32, (16,), 0)
    zv = jnp.zeros((16,), jnp.int32)

    with jax.named_scope("ph_dma_in"):
        pltpu.sync_copy(top1_hbm.at[pl.ds(base, CHUNK)], idx_v)
        pltpu.sync_copy(wgt_hbm.at[pl.ds(base, CHUNK)], wgt_v)

    for j in range(ZCHUNK // 16):
        zi_v[pl.ds(j * 16, 16)] = (iota + (s * ZCHUNK + j * 16)) & (S - 1)
        zf_v[pl.ds(j * 16, 16)] = jnp.zeros((16,), jnp.float32)
    for j in range(CHUNK // 16):
        tok_v[pl.ds(j * 16, 16)] = iota + (base + j * 16)

    sc_hist = jax.named_scope("ph_hist")
    sc_hist.__enter__()

    def hist_body(i, hq):
        ev = idx_v[pl.ds(i * 16, 16)]
        for l in range(16):
            e = ev[l]
            hq = tuple(hq[q] + (iota + 16 * q == e).astype(jnp.int32)
                       for q in range(E // 16))
        return hq
    hq = lax.fori_loop(0, CHUNK // 16, hist_body, (zv,) * (E // 16))
    for q in range(E // 16):
        hist_v[pl.ds(q * 16, 16)] = hq[q]
    sc_hist.__exit__(None, None, None)

    sc_pub = jax.named_scope("ph_publish")
    sc_pub.__enter__()
    pltpu.sync_copy(hist_v, hist_sh.at[pl.ds(s * E, E)])
    pltpu.sync_copy(zi_v, ids_sh.at[pl.ds(s * ZCHUNK, ZCHUNK)])
    pltpu.sync_copy(zf_v, ws_sh.at[pl.ds(s * ZCHUNK, ZCHUNK)])
    plsc.subcore_barrier()
    pltpu.sync_copy(hist_sh, allhist_v)
    sc_pub.__exit__(None, None, None)

    sc_meta = jax.named_scope("ph_meta")
    sc_meta.__enter__()
    for q in range((E + 16) // 16):
        seglist_v[pl.ds(q * 16, 16)] = zv
    carry = jnp.int32(0)
    segcarry = jnp.int32(0)
    for q in range(E // 16):
        tot = zv
        rb = zv
        for t in range(NS):
            h = allhist_v[pl.ds(t * E + q * 16, 16)]
            tot = tot + h
            rb = rb + jnp.where(jnp.full((16,), t, jnp.int32) < s, h, zv)
        rb_v[pl.ds(q * 16, 16)] = rb
        nblk = (tot + (BLOCK - 1)) >> 5
        ends = plsc.cumsum(nblk) + carry
        ends_v[pl.ds(q * 16, 16)] = ends
        po_v[pl.ds(q * 16, 16)] = (ends - nblk) * BLOCK
        carry = carry + jnp.sum(nblk)
        nz = jnp.minimum(nblk, 1)
        segrank = plsc.cumsum(nz) - nz + segcarry
        plsc.store_scatter(seglist_v, [segrank], iota + 16 * q,
                           mask=(nz == 1))
        segcarry = segcarry + jnp.sum(nz)
    seglist_v[pl.ds(E, 16)] = zv + segcarry

    sc_meta.__exit__(None, None, None)
    sc_pos = jax.named_scope("ph_pos")
    sc_pos.__enter__()
    def pos_body(i, lcq):
        for q in range(E // 16):
            runh_v[pl.ds(q * 16, 16)] = lcq[q]
        ev = idx_v[pl.ds(i * 16, 16)]
        pv = (plsc.load_gather(po_v, [ev]) + plsc.load_gather(rb_v, [ev])
              + plsc.load_gather(runh_v, [ev]))
        intra = zv
        for l in range(16):
            e = ev[l]
            intra = intra + ((iota > l) & (ev == e)).astype(jnp.int32)
            lcq = tuple(lcq[q] + (iota + 16 * q == e).astype(jnp.int32)
                        for q in range(E // 16))
        plist_v[pl.ds(i * 16, 16)] = pv + intra
        return lcq
    lax.fori_loop(0, CHUNK // 16, pos_body, (zv,) * (E // 16))
    sc_pos.__exit__(None, None, None)

    with jax.named_scope("ph_scatter"):
        plsc.subcore_barrier()
        pltpu.sync_copy(tok_v, ids_sh.at[plist_v])
        pltpu.sync_copy(wgt_v, ws_sh.at[plist_v])
        plsc.subcore_barrier()

    sc_be = jax.named_scope("ph_be")
    sc_be.__enter__()

    def be_body(i, carry2):
        gv = iota + i * 16
        cnt = zv
        for q in range(E // 16):
            endsq = ends_v[pl.ds(q * 16, 16)]
            for l in range(16):
                cnt = cnt + (gv >= endsq[l]).astype(jnp.int32)
        be_v[pl.ds(i * 16, 16)] = jnp.minimum(cnt, E - 1)
        return carry2
    lax.fori_loop(0, G // 16, be_body, 0)

    @pl.when(jnp.logical_and(s == 0, c == 0))
    def _():
        pltpu.sync_copy(be_v, be_hbm)
        pltpu.sync_copy(seglist_v, seg_hbm)

    @pl.when(c == 0)
    def _():
        pltpu.sync_copy(plist_v, pos_hbm.at[pl.ds(base, CHUNK)])
        pltpu.sync_copy(ws_sh.at[pl.ds(s * ZCHUNK, ZCHUNK)],
                        ws_hbm.at[pl.ds(s * ZCHUNK, ZCHUNK)])

    sc_be.__exit__(None, None, None)

    with jax.named_scope("ph_xgather"):
        w = s * NC + c
        for r in range(2):
            row0 = w * PW + r * HPW
            pltpu.sync_copy(ids_sh.at[pl.ds(row0, HPW)], gid_v)
            pltpu.async_copy(x_hbm.at[gid_v], rows_v, sem).wait()
            pltpu.sync_copy(rows_v, xs_hbm.at[pl.ds(row0, HPW)])


@functools.partial(
    pl.kernel,
    out_type=jax.ShapeDtypeStruct((S, DIM), jnp.float32),
    mesh=_sc_mesh,
    scratch_types=[
        pltpu.VMEM((UCHUNK,), jnp.int32),
        pltpu.VMEM((UCHUNK, DIM), jnp.float32),
        pltpu.SemaphoreType.DMA,
    ],
    compiler_params=pltpu.CompilerParams(needs_layout_passes=False),
)
def _sc_unpermute(pos_hbm, ys_hbm, out_hbm, pid_v, rows_v, sem):
    w = lax.axis_index("s") * NC + lax.axis_index("c")
    base = w * UCHUNK
    pltpu.sync_copy(pos_hbm.at[pl.ds(base, UCHUNK)], pid_v)
    pltpu.async_copy(ys_hbm.at[pid_v], rows_v, sem).wait()
    pltpu.sync_copy(rows_v, out_hbm.at[pl.ds(base, UCHUNK)])


@jax.jit
def kernel(x, gate_w, w1, w2, balance_bias):
    b, s, d = x.shape
    xf = x.reshape(s, d)

    top1, wgt = pl.pallas_call(
        _gate_kernel,
        grid=(s // TB,),
        in_specs=[
            pl.BlockSpec((TB, DIM), lambda t: (t, 0)),
            pl.BlockSpec((E, DIM), lambda t: (0, 0)),
            pl.BlockSpec((1, E), lambda t: (0, 0)),
        ],
        out_specs=[
            pl.BlockSpec((TB, 1), lambda t: (t, 0)),
            pl.BlockSpec((TB, 1), lambda t: (t, 0)),
        ],
        out_shape=[
            jax.ShapeDtypeStruct((s, 1), jnp.int32),
            jax.ShapeDtypeStruct((s, 1), jnp.float32),
        ],
    )(xf, gate_w, balance_bias.reshape(1, E))

    x_sorted, ws, block_expert, pos, seglist = _sc_route(
        top1.reshape(s), wgt.reshape(s), xf)

    y_sorted = pl.pallas_call(
        _ffn_kernel,
        grid_spec=pltpu.PrefetchScalarGridSpec(
            num_scalar_prefetch=2,
            grid=(1,),
            in_specs=[
                pl.BlockSpec(memory_space=pl.ANY),
                pl.BlockSpec(memory_space=pl.ANY),
                pl.BlockSpec(memory_space=pl.ANY),
                pl.BlockSpec(memory_space=pl.ANY),
            ],
            out_specs=pl.BlockSpec(memory_space=pl.ANY),
            scratch_shapes=[
                pltpu.VMEM((P, DIM), jnp.float32),
                pltpu.VMEM((P, DIM), jnp.float32),
                pltpu.VMEM((P, 1), jnp.float32),
                pltpu.VMEM((4, DIM, ED), jnp.float32),
                pltpu.VMEM((4, ED, DIM), jnp.float32),
                pltpu.SemaphoreType.DMA((NCH,)),
                pltpu.SemaphoreType.DMA,
                pltpu.SemaphoreType.DMA((4, 2)),
                pltpu.SemaphoreType.DMA,
            ],
        ),
        out_shape=jax.ShapeDtypeStruct((P, DIM), jnp.float32),
        compiler_params=pltpu.CompilerParams(
            vmem_limit_bytes=100 * 1024 * 1024),
    )(block_expert, seglist, x_sorted, w1, w2, ws.reshape(P, 1))

    out = _sc_unpermute(pos, y_sorted)
    return out.reshape(b, s, d)


# speedup vs baseline: 1.0093x; 1.0093x over previous
import functools

import jax
import jax.numpy as jnp
from jax import lax
from jax.experimental import pallas as pl
from jax.experimental.pallas import tpu as pltpu
from jax.experimental.pallas import tpu_sc as plsc

S = 2048
DIM = 768
E = 64
ED = 128
BLOCK = 32
G = S // BLOCK + E
P = G * BLOCK
TB = 256

NC, NS = 2, 16
NW = NC * NS
CHUNK = S // NS
ZCHUNK = P // NS
PW = P // NW
HPW = PW // 2
UCHUNK = S // NW


def _gate_kernel(x_ref, gw_ref, bias_ref, idx_ref, wgt_ref):
    logits = jax.lax.dot_general(
        x_ref[...], gw_ref[...], (((1,), (1,)), ((), ())),
        preferred_element_type=jnp.float32,
    )
    scores = jax.nn.sigmoid(logits + bias_ref[...])
    m = jnp.max(scores, axis=1, keepdims=True)
    eids = jax.lax.broadcasted_iota(jnp.int32, scores.shape, 1)
    idx_ref[...] = jnp.min(jnp.where(scores == m, eids, E), axis=1, keepdims=True)
    wgt_ref[...] = m / (m + 1e-6)


NCH = 8
CROWS = P // NCH
CBLK = CROWS // BLOCK


def _ffn_kernel(be_ref, seg_ref, xs_hbm, w1_hbm, w2_hbm, ws_hbm, y_hbm,
                xsv, yv, wsv, w1b, w2b, xsems, wsem, w3sems, osem):
    nseg = seg_ref[E]

    def _wcopies(e_, slot_):
        return (
            pltpu.make_async_copy(w1_hbm.at[e_, pl.ds(0, DIM // 2)],
                                  w1b.at[slot_, pl.ds(0, DIM // 2)],
                                  w3sems.at[slot_, 0]),
            pltpu.make_async_copy(w1_hbm.at[e_, pl.ds(DIM // 2, DIM // 2)],
                                  w1b.at[slot_, pl.ds(DIM // 2, DIM // 2)],
                                  w3sems.at[slot_, 1]),
            pltpu.make_async_copy(w2_hbm.at[e_, pl.ds(0, ED // 2)],
                                  w2b.at[slot_, pl.ds(0, ED // 2)],
                                  w3sems.at[slot_, 2]),
            pltpu.make_async_copy(w2_hbm.at[e_, pl.ds(ED // 2, ED // 2)],
                                  w2b.at[slot_, pl.ds(ED // 2, ED // 2)],
                                  w3sems.at[slot_, 3]),
        )

    for ch in range(NCH):
        pltpu.make_async_copy(xs_hbm.at[pl.ds(ch * CROWS, CROWS)],
                              xsv.at[pl.ds(ch * CROWS, CROWS)],
                              xsems.at[ch]).start()
    pltpu.make_async_copy(ws_hbm, wsv, wsem).start()
    for cp in _wcopies(seg_ref[0], 0):
        cp.start()

    @pl.when(nseg > 1)
    def _():
        for cp in _wcopies(seg_ref[1], 1):
            cp.start()

    pltpu.make_async_copy(ws_hbm, wsv, wsem).wait()

    def gloop(g, carry):
        k, s3 = carry
        e = be_ref[g]
        prv = be_ref[jnp.maximum(g - 1, 0)]
        is_new = jnp.logical_or(g == 0, e != prv)
        k = jnp.where(is_new, k + 1, k)
        s3 = jnp.where(is_new, jnp.where(s3 == 2, 0, s3 + 1), s3)

        @pl.when((g & (CBLK - 1)) == 0)
        def _():
            ch = g >> 4
            pltpu.make_async_copy(xs_hbm.at[pl.ds(ch * CROWS, CROWS)],
                                  xsv.at[pl.ds(ch * CROWS, CROWS)],
                                  xsems.at[ch]).wait()

        @pl.when(jnp.logical_and(is_new, k < nseg))
        def _():
            for cp in _wcopies(e, s3):
                cp.wait()
            nk = k + 2

            @pl.when(nk < nseg)
            def _():
                tgt = jnp.where(s3 == 0, 2, s3 - 1)
                for cp in _wcopies(seg_ref[nk], tgt):
                    cp.start()

        xblk = xsv[pl.ds(g * BLOCK, BLOCK), :]
        h = jnp.dot(xblk, w1b[s3], preferred_element_type=jnp.float32)
        h = h * jax.nn.sigmoid(h)
        y = jnp.dot(h, w2b[s3], preferred_element_type=jnp.float32)
        yv[pl.ds(g * BLOCK, BLOCK), :] = y * wsv[pl.ds(g * BLOCK, BLOCK), :]

        @pl.when((g & (CBLK - 1)) == (CBLK - 1))
        def _():
            ch = g >> 4
            pltpu.make_async_copy(yv.at[pl.ds(ch * CROWS, CROWS)],
                                  y_hbm.at[pl.ds(ch * CROWS, CROWS)],
                                  osem).start()
        return (k, s3)

    lax.fori_loop(0, G, gloop, (jnp.int32(-1), jnp.int32(2)))
    for ch in range(NCH):
        pltpu.make_async_copy(yv.at[pl.ds(ch * CROWS, CROWS)],
                              y_hbm.at[pl.ds(ch * CROWS, CROWS)],
                              osem).wait()


_sc_mesh = plsc.VectorSubcoreMesh(core_axis_name="c", subcore_axis_name="s")


@functools.partial(
    pl.kernel,
    out_type=[
        jax.ShapeDtypeStruct((P, DIM), jnp.float32),
        jax.ShapeDtypeStruct((P,), jnp.float32),
        jax.ShapeDtypeStruct((G,), jnp.int32),
        jax.ShapeDtypeStruct((S,), jnp.int32),
        jax.ShapeDtypeStruct((E + 16,), jnp.int32),
    ],
    mesh=_sc_mesh,
    scratch_types=[
        pltpu.VMEM((CHUNK,), jnp.int32),
        pltpu.VMEM((CHUNK,), jnp.float32),
        pltpu.VMEM((NS * E,), jnp.int32),
        pltpu.VMEM((E,), jnp.int32),
        pltpu.VMEM((E,), jnp.int32),
        pltpu.VMEM((E,), jnp.int32),
        pltpu.VMEM((E,), jnp.int32),
        pltpu.VMEM((E,), jnp.int32),
        pltpu.VMEM((E + 16,), jnp.int32),
        pltpu.VMEM((CHUNK,), jnp.int32),
        pltpu.VMEM((CHUNK,), jnp.int32),
        pltpu.VMEM((G,), jnp.int32),
        pltpu.VMEM((ZCHUNK,), jnp.int32),
        pltpu.VMEM((ZCHUNK,), jnp.float32),
        pltpu.VMEM((HPW,), jnp.int32),
        pltpu.VMEM((HPW, DIM), jnp.float32),
        pltpu.VMEM_SHARED((NS * E,), jnp.int32),
        pltpu.VMEM_SHARED((P,), jnp.int32),
        pltpu.VMEM_SHARED((P,), jnp.float32),
        pltpu.SemaphoreType.DMA,
    ],
    compiler_params=pltpu.CompilerParams(needs_layout_passes=False),
)
def _sc_route(top1_hbm, wgt_hbm, x_hbm, xs_hbm, ws_hbm, be_hbm, pos_hbm,
              seg_hbm, idx_v, wgt_v, allhist_v, hist_v, rb_v, po_v, runh_v,
              ends_v, seglist_v, plist_v, tok_v, be_v, zi_v, zf_v,
              gid_v, rows_v, hist_sh, ids_sh, ws_sh, sem):
    c = lax.axis_index("c")
    s = lax.axis_index("s")
    base = s * CHUNK
    iota = lax.broadcasted_iota(jnp.int32, (16,), 0)
    zv = jnp.zeros((16,), jnp.int32)

    with jax.named_scope("ph_dma_in"):
        pltpu.sync_copy(top1_hbm.at[pl.ds(base, CHUNK)], idx_v)
        pltpu.sync_copy(wgt_hbm.at[pl.ds(base, CHUNK)], wgt_v)

    for j in range(ZCHUNK // 16):
        zi_v[pl.ds(j * 16, 16)] = (iota + (s * ZCHUNK + j * 16)) & (S - 1)
        zf_v[pl.ds(j * 16, 16)] = jnp.zeros((16,), jnp.float32)
    for j in range(CHUNK // 16):
        tok_v[pl.ds(j * 16, 16)] = iota + (base + j * 16)

    sc_hist = jax.named_scope("ph_hist")
    sc_hist.__enter__()

    def hist_body(i, hq):
        ev = idx_v[pl.ds(i * 16, 16)]
        for l in range(16):
            e = ev[l]
            hq = tuple(hq[q] + (iota + 16 * q == e).astype(jnp.int32)
                       for q in range(E // 16))
        return hq
    hq = lax.fori_loop(0, CHUNK // 16, hist_body, (zv,) * (E // 16))
    for q in range(E // 16):
        hist_v[pl.ds(q * 16, 16)] = hq[q]
    sc_hist.__exit__(None, None, None)

    sc_pub = jax.named_scope("ph_publish")
    sc_pub.__enter__()
    pltpu.sync_copy(hist_v, hist_sh.at[pl.ds(s * E, E)])
    pltpu.sync_copy(zi_v, ids_sh.at[pl.ds(s * ZCHUNK, ZCHUNK)])
    pltpu.sync_copy(zf_v, ws_sh.at[pl.ds(s * ZCHUNK, ZCHUNK)])
    plsc.subcore_barrier()
    pltpu.sync_copy(hist_sh, allhist_v)
    sc_pub.__exit__(None, None, None)

    sc_meta = jax.named_scope("ph_meta")
    sc_meta.__enter__()
    for q in range((E + 16) // 16):
        seglist_v[pl.ds(q * 16, 16)] = zv
    carry = jnp.int32(0)
    segcarry = jnp.int32(0)
    for q in range(E // 16):
        tot = zv
        rb = zv
        for t in range(NS):
            h = allhist_v[pl.ds(t * E + q * 16, 16)]
            tot = tot + h
            rb = rb + jnp.where(jnp.full((16,), t, jnp.int32) < s, h, zv)
        rb_v[pl.ds(q * 16, 16)] = rb
        nblk = (tot + (BLOCK - 1)) >> 5
        ends = plsc.cumsum(nblk) + carry
        ends_v[pl.ds(q * 16, 16)] = ends
        po_v[pl.ds(q * 16, 16)] = (ends - nblk) * BLOCK
        carry = carry + jnp.sum(nblk)
        nz = jnp.minimum(nblk, 1)
        segrank = plsc.cumsum(nz) - nz + segcarry
        plsc.store_scatter(seglist_v, [segrank], iota + 16 * q,
                           mask=(nz == 1))
        segcarry = segcarry + jnp.sum(nz)
    seglist_v[pl.ds(E, 16)] = zv + segcarry

    sc_meta.__exit__(None, None, None)
    sc_pos = jax.named_scope("ph_pos")
    sc_pos.__enter__()
    def pos_body(i, lcq):
        for q in range(E // 16):
            runh_v[pl.ds(q * 16, 16)] = lcq[q]
        ev = idx_v[pl.ds(i * 16, 16)]
        pv = (plsc.load_gather(po_v, [ev]) + plsc.load_gather(rb_v, [ev])
              + plsc.load_gather(runh_v, [ev]))
        intra = zv
        for l in range(16):
            e = ev[l]
            intra = intra + ((iota > l) & (ev == e)).astype(jnp.int32)
            lcq = tuple(lcq[q] + (iota + 16 * q == e).astype(jnp.int32)
                        for q in range(E // 16))
        plist_v[pl.ds(i * 16, 16)] = pv + intra
        return lcq
    lax.fori_loop(0, CHUNK // 16, pos_body, (zv,) * (E // 16))
    sc_pos.__exit__(None, None, None)

    with jax.named_scope("ph_scatter"):
        plsc.subcore_barrier()
        pltpu.sync_copy(tok_v, ids_sh.at[plist_v])
        pltpu.sync_copy(wgt_v, ws_sh.at[plist_v])
        plsc.subcore_barrier()

    sc_be = jax.named_scope("ph_be")
    sc_be.__enter__()

    def be_body(i, carry2):
        gv = iota + i * 16
        cnt = zv
        for q in range(E // 16):
            endsq = ends_v[pl.ds(q * 16, 16)]
            for l in range(16):
                cnt = cnt + (gv >= endsq[l]).astype(jnp.int32)
        be_v[pl.ds(i * 16, 16)] = jnp.minimum(cnt, E - 1)
        return carry2
    lax.fori_loop(0, G // 16, be_body, 0)

    @pl.when(jnp.logical_and(s == 0, c == 0))
    def _():
        pltpu.sync_copy(be_v, be_hbm)
        pltpu.sync_copy(seglist_v, seg_hbm)

    @pl.when(c == 0)
    def _():
        pltpu.sync_copy(plist_v, pos_hbm.at[pl.ds(base, CHUNK)])
        pltpu.sync_copy(ws_sh.at[pl.ds(s * ZCHUNK, ZCHUNK)],
                        ws_hbm.at[pl.ds(s * ZCHUNK, ZCHUNK)])

    sc_be.__exit__(None, None, None)

    with jax.named_scope("ph_xgather"):
        w = s * NC + c
        for r in range(2):
            row0 = w * PW + r * HPW
            pltpu.sync_copy(ids_sh.at[pl.ds(row0, HPW)], gid_v)
            pltpu.async_copy(x_hbm.at[gid_v], rows_v, sem).wait()
            pltpu.sync_copy(rows_v, xs_hbm.at[pl.ds(row0, HPW)])


@functools.partial(
    pl.kernel,
    out_type=jax.ShapeDtypeStruct((S, DIM), jnp.float32),
    mesh=_sc_mesh,
    scratch_types=[
        pltpu.VMEM((UCHUNK,), jnp.int32),
        pltpu.VMEM((UCHUNK, DIM), jnp.float32),
        pltpu.SemaphoreType.DMA,
    ],
    compiler_params=pltpu.CompilerParams(needs_layout_passes=False),
)
def _sc_unpermute(pos_hbm, ys_hbm, out_hbm, pid_v, rows_v, sem):
    w = lax.axis_index("s") * NC + lax.axis_index("c")
    base = w * UCHUNK
    pltpu.sync_copy(pos_hbm.at[pl.ds(base, UCHUNK)], pid_v)
    pltpu.async_copy(ys_hbm.at[pid_v], rows_v, sem).wait()
    pltpu.sync_copy(rows_v, out_hbm.at[pl.ds(base, UCHUNK)])


@jax.jit
def kernel(x, gate_w, w1, w2, balance_bias):
    b, s, d = x.shape
    xf = x.reshape(s, d)

    top1, wgt = pl.pallas_call(
        _gate_kernel,
        grid=(s // TB,),
        in_specs=[
            pl.BlockSpec((TB, DIM), lambda t: (t, 0)),
            pl.BlockSpec((E, DIM), lambda t: (0, 0)),
            pl.BlockSpec((1, E), lambda t: (0, 0)),
        ],
        out_specs=[
            pl.BlockSpec((TB, 1), lambda t: (t, 0)),
            pl.BlockSpec((TB, 1), lambda t: (t, 0)),
        ],
        out_shape=[
            jax.ShapeDtypeStruct((s, 1), jnp.int32),
            jax.ShapeDtypeStruct((s, 1), jnp.float32),
        ],
    )(xf, gate_w, balance_bias.reshape(1, E))

    x_sorted, ws, block_expert, pos, seglist = _sc_route(
        top1.reshape(s), wgt.reshape(s), xf)

    y_sorted = pl.pallas_call(
        _ffn_kernel,
        grid_spec=pltpu.PrefetchScalarGridSpec(
            num_scalar_prefetch=2,
            grid=(1,),
            in_specs=[
                pl.BlockSpec(memory_space=pl.ANY),
                pl.BlockSpec(memory_space=pl.ANY),
                pl.BlockSpec(memory_space=pl.ANY),
                pl.BlockSpec(memory_space=pl.ANY),
            ],
            out_specs=pl.BlockSpec(memory_space=pl.ANY),
            scratch_shapes=[
                pltpu.VMEM((P, DIM), jnp.float32),
                pltpu.VMEM((P, DIM), jnp.float32),
                pltpu.VMEM((P, 1), jnp.float32),
                pltpu.VMEM((3, DIM, ED), jnp.float32),
                pltpu.VMEM((3, ED, DIM), jnp.float32),
                pltpu.SemaphoreType.DMA((NCH,)),
                pltpu.SemaphoreType.DMA,
                pltpu.SemaphoreType.DMA((3, 4)),
                pltpu.SemaphoreType.DMA,
            ],
        ),
        out_shape=jax.ShapeDtypeStruct((P, DIM), jnp.float32),
        compiler_params=pltpu.CompilerParams(
            vmem_limit_bytes=100 * 1024 * 1024),
    )(block_expert, seglist, x_sorted, w1, w2, ws.reshape(P, 1))

    out = _sc_unpermute(pos, y_sorted)
    return out.reshape(b, s, d)


# speedup vs baseline: 1.0123x; 1.0029x over previous
import functools

import jax
import jax.numpy as jnp
from jax import lax
from jax.experimental import pallas as pl
from jax.experimental.pallas import tpu as pltpu
from jax.experimental.pallas import tpu_sc as plsc

S = 2048
DIM = 768
E = 64
ED = 128
BLOCK = 32
G = S // BLOCK + E
P = G * BLOCK
TB = 256

NC, NS = 2, 16
NW = NC * NS
CHUNK = S // NS
ZCHUNK = P // NS
PW = P // NW
HPW = PW // 2
UCHUNK = S // NW


def _gate_kernel(x_ref, gw_ref, bias_ref, idx_ref, wgt_ref):
    logits = jax.lax.dot_general(
        x_ref[...], gw_ref[...], (((1,), (1,)), ((), ())),
        preferred_element_type=jnp.float32,
    )
    scores = jax.nn.sigmoid(logits + bias_ref[...])
    m = jnp.max(scores, axis=1, keepdims=True)
    eids = jax.lax.broadcasted_iota(jnp.int32, scores.shape, 1)
    idx_ref[...] = jnp.min(jnp.where(scores == m, eids, E), axis=1, keepdims=True)
    wgt_ref[...] = m / (m + 1e-6)


NCH = 8
CROWS = P // NCH
CBLK = CROWS // BLOCK


def _ffn_kernel(be_ref, seg_ref, xs_hbm, w1_hbm, w2_hbm, ws_hbm, y_hbm,
                xsv, yv, wsv, w1b, w2b, xsems, wsem, w3sems, osem):
    nseg = seg_ref[E]
    for ch in range(NCH):
        pltpu.make_async_copy(xs_hbm.at[pl.ds(ch * CROWS, CROWS)],
                              xsv.at[pl.ds(ch * CROWS, CROWS)],
                              xsems.at[ch]).start()
    pltpu.make_async_copy(ws_hbm, wsv, wsem).start()
    e0 = seg_ref[0]
    pltpu.make_async_copy(w1_hbm.at[e0], w1b.at[0], w3sems.at[0, 0]).start()
    pltpu.make_async_copy(w2_hbm.at[e0], w2b.at[0], w3sems.at[0, 1]).start()

    @pl.when(nseg > 1)
    def _():
        e1 = seg_ref[1]
        pltpu.make_async_copy(w1_hbm.at[e1], w1b.at[1], w3sems.at[1, 0]).start()
        pltpu.make_async_copy(w2_hbm.at[e1], w2b.at[1], w3sems.at[1, 1]).start()

    pltpu.make_async_copy(ws_hbm, wsv, wsem).wait()

    def gloop(g, carry):
        k, s3 = carry
        e = be_ref[g]
        prv = be_ref[jnp.maximum(g - 1, 0)]
        is_new = jnp.logical_or(g == 0, e != prv)
        k = jnp.where(is_new, k + 1, k)
        s3 = jnp.where(is_new, jnp.where(s3 == 2, 0, s3 + 1), s3)

        @pl.when((g & (CBLK - 1)) == 0)
        def _():
            ch = g >> 4
            pltpu.make_async_copy(xs_hbm.at[pl.ds(ch * CROWS, CROWS)],
                                  xsv.at[pl.ds(ch * CROWS, CROWS)],
                                  xsems.at[ch]).wait()

        @pl.when(jnp.logical_and(is_new, k < nseg))
        def _():
            pltpu.make_async_copy(w1_hbm.at[e], w1b.at[s3],
                                  w3sems.at[s3, 0]).wait()
            pltpu.make_async_copy(w2_hbm.at[e], w2b.at[s3],
                                  w3sems.at[s3, 1]).wait()
            nk = k + 2

            @pl.when(nk < nseg)
            def _():
                tgt = jnp.where(s3 == 0, 2, s3 - 1)
                ne = seg_ref[nk]
                pltpu.make_async_copy(w1_hbm.at[ne], w1b.at[tgt],
                                      w3sems.at[tgt, 0]).start()
                pltpu.make_async_copy(w2_hbm.at[ne], w2b.at[tgt],
                                      w3sems.at[tgt, 1]).start()

        xblk = xsv[pl.ds(g * BLOCK, BLOCK), :]
        h = jnp.dot(xblk, w1b[s3], preferred_element_type=jnp.float32)
        h = h * jax.nn.sigmoid(h)
        y = jnp.dot(h, w2b[s3], preferred_element_type=jnp.float32)
        yv[pl.ds(g * BLOCK, BLOCK), :] = y * wsv[pl.ds(g * BLOCK, BLOCK), :]

        @pl.when((g & (CBLK - 1)) == (CBLK - 1))
        def _():
            ch = g >> 4
            pltpu.make_async_copy(yv.at[pl.ds(ch * CROWS, CROWS)],
                                  y_hbm.at[pl.ds(ch * CROWS, CROWS)],
                                  osem).start()
        return (k, s3)

    lax.fori_loop(0, G, gloop, (jnp.int32(-1), jnp.int32(2)))
    for ch in range(NCH):
        pltpu.make_async_copy(yv.at[pl.ds(ch * CROWS, CROWS)],
                              y_hbm.at[pl.ds(ch * CROWS, CROWS)],
                              osem).wait()


_sc_mesh = plsc.VectorSubcoreMesh(core_axis_name="c", subcore_axis_name="s")


@functools.partial(
    pl.kernel,
    out_type=[
        jax.ShapeDtypeStruct((P, DIM), jnp.float32),
        jax.ShapeDtypeStruct((P,), jnp.float32),
        jax.ShapeDtypeStruct((G,), jnp.int32),
        jax.ShapeDtypeStruct((S,), jnp.int32),
        jax.ShapeDtypeStruct((E + 16,), jnp.int32),
    ],
    mesh=_sc_mesh,
    scratch_types=[
        pltpu.VMEM((CHUNK,), jnp.int32),
        pltpu.VMEM((CHUNK,), jnp.float32),
        pltpu.VMEM((NS * E,), jnp.int32),
        pltpu.VMEM((E,), jnp.int32),
        pltpu.VMEM((E,), jnp.int32),
        pltpu.VMEM((E,), jnp.int32),
        pltpu.VMEM((E,), jnp.int32),
        pltpu.VMEM((E,), jnp.int32),
        pltpu.VMEM((E + 16,), jnp.int32),
        pltpu.VMEM((CHUNK,), jnp.int32),
        pltpu.VMEM((CHUNK,), jnp.int32),
        pltpu.VMEM((G,), jnp.int32),
        pltpu.VMEM((ZCHUNK,), jnp.int32),
        pltpu.VMEM((ZCHUNK,), jnp.float32),
        pltpu.VMEM((HPW,), jnp.int32),
        pltpu.VMEM((HPW, DIM), jnp.float32),
        pltpu.VMEM_SHARED((NS * E,), jnp.int32),
        pltpu.VMEM_SHARED((P,), jnp.int32),
        pltpu.VMEM_SHARED((P,), jnp.float32),
        pltpu.SemaphoreType.DMA,
    ],
    compiler_params=pltpu.CompilerParams(needs_layout_passes=False),
)
def _sc_route(top1_hbm, wgt_hbm, x_hbm, xs_hbm, ws_hbm, be_hbm, pos_hbm,
              seg_hbm, idx_v, wgt_v, allhist_v, hist_v, rb_v, po_v, runh_v,
              ends_v, seglist_v, plist_v, tok_v, be_v, zi_v, zf_v,
              gid_v, rows_v, hist_sh, ids_sh, ws_sh, sem):
    c = lax.axis_index("c")
    s = lax.axis_index("s")
    base = s * CHUNK
    iota = lax.broadcasted_iota(jnp.int32, (16,), 0)
    zv = jnp.zeros((16,), jnp.int32)

    with jax.named_scope("ph_dma_in"):
        pltpu.sync_copy(top1_hbm.at[pl.ds(base, CHUNK)], idx_v)
        pltpu.sync_copy(wgt_hbm.at[pl.ds(base, CHUNK)], wgt_v)

    for j in range(ZCHUNK // 16):
        zi_v[pl.ds(j * 16, 16)] = (iota + (s * ZCHUNK + j * 16)) & (S - 1)
        zf_v[pl.ds(j * 16, 16)] = jnp.zeros((16,), jnp.float32)
    for j in range(CHUNK // 16):
        tok_v[pl.ds(j * 16, 16)] = iota + (base + j * 16)

    sc_hist = jax.named_scope("ph_hist")
    sc_hist.__enter__()

    def hist_body(i, hq):
        ev = idx_v[pl.ds(i * 16, 16)]
        for l in range(16):
            e = ev[l]
            hq = tuple(hq[q] + (iota + 16 * q == e).astype(jnp.int32)
                       for q in range(E // 16))
        return hq
    hq = lax.fori_loop(0, CHUNK // 16, hist_body, (zv,) * (E // 16))
    for q in range(E // 16):
        hist_v[pl.ds(q * 16, 16)] = hq[q]
    sc_hist.__exit__(None, None, None)

    sc_pub = jax.named_scope("ph_publish")
    sc_pub.__enter__()
    pltpu.sync_copy(hist_v, hist_sh.at[pl.ds(s * E, E)])
    pltpu.sync_copy(zi_v, ids_sh.at[pl.ds(s * ZCHUNK, ZCHUNK)])
    pltpu.sync_copy(zf_v, ws_sh.at[pl.ds(s * ZCHUNK, ZCHUNK)])
    plsc.subcore_barrier()
    pltpu.sync_copy(hist_sh, allhist_v)
    sc_pub.__exit__(None, None, None)

    sc_meta = jax.named_scope("ph_meta")
    sc_meta.__enter__()
    for q in range((E + 16) // 16):
        seglist_v[pl.ds(q * 16, 16)] = zv
    carry = jnp.int32(0)
    segcarry = jnp.int32(0)
    for q in range(E // 16):
        tot = zv
        rb = zv
        for t in range(NS):
            h = allhist_v[pl.ds(t * E + q * 16, 16)]
            tot = tot + h
            rb = rb + jnp.where(jnp.full((16,), t, jnp.int32) < s, h, zv)
        rb_v[pl.ds(q * 16, 16)] = rb
        nblk = (tot + (BLOCK - 1)) >> 5
        ends = plsc.cumsum(nblk) + carry
        ends_v[pl.ds(q * 16, 16)] = ends
        po_v[pl.ds(q * 16, 16)] = (ends - nblk) * BLOCK
        carry = carry + jnp.sum(nblk)
        nz = jnp.minimum(nblk, 1)
        segrank = plsc.cumsum(nz) - nz + segcarry
        plsc.store_scatter(seglist_v, [segrank], iota + 16 * q,
                           mask=(nz == 1))
        segcarry = segcarry + jnp.sum(nz)
    seglist_v[pl.ds(E, 16)] = zv + segcarry

    sc_meta.__exit__(None, None, None)
    sc_pos = jax.named_scope("ph_pos")
    sc_pos.__enter__()
    def pos_body(i, lcq):
        for q in range(E // 16):
            runh_v[pl.ds(q * 16, 16)] = lcq[q]
        ev = idx_v[pl.ds(i * 16, 16)]
        pv = (plsc.load_gather(po_v, [ev]) + plsc.load_gather(rb_v, [ev])
              + plsc.load_gather(runh_v, [ev]))
        intra = zv
        for l in range(16):
            e = ev[l]
            intra = intra + ((iota > l) & (ev == e)).astype(jnp.int32)
            lcq = tuple(lcq[q] + (iota + 16 * q == e).astype(jnp.int32)
                        for q in range(E // 16))
        plist_v[pl.ds(i * 16, 16)] = pv + intra
        return lcq
    lax.fori_loop(0, CHUNK // 16, pos_body, (zv,) * (E // 16))
    sc_pos.__exit__(None, None, None)

    with jax.named_scope("ph_scatter"):
        plsc.subcore_barrier()
        pltpu.sync_copy(tok_v, ids_sh.at[plist_v])
        pltpu.sync_copy(wgt_v, ws_sh.at[plist_v])
        plsc.subcore_barrier()

    sc_be = jax.named_scope("ph_be")
    sc_be.__enter__()

    def be_body(i, carry2):
        gv = iota + i * 16
        cnt = zv
        for q in range(E // 16):
            endsq = ends_v[pl.ds(q * 16, 16)]
            for l in range(16):
                cnt = cnt + (gv >= endsq[l]).astype(jnp.int32)
        be_v[pl.ds(i * 16, 16)] = jnp.minimum(cnt, E - 1)
        return carry2
    lax.fori_loop(0, G // 16, be_body, 0)

    @pl.when(jnp.logical_and(s == 0, c == 0))
    def _():
        pltpu.sync_copy(be_v, be_hbm)
        pltpu.sync_copy(seglist_v, seg_hbm)

    @pl.when(c == 0)
    def _():
        pltpu.sync_copy(plist_v, pos_hbm.at[pl.ds(base, CHUNK)])
        pltpu.sync_copy(ws_sh.at[pl.ds(s * ZCHUNK, ZCHUNK)],
                        ws_hbm.at[pl.ds(s * ZCHUNK, ZCHUNK)])

    sc_be.__exit__(None, None, None)

    with jax.named_scope("ph_xgather"):
        w = s * NC + c
        for r in range(2):
            row0 = w * PW + r * HPW
            pltpu.sync_copy(ids_sh.at[pl.ds(row0, HPW)], gid_v)
            pltpu.async_copy(x_hbm.at[gid_v], rows_v, sem).wait()
            pltpu.sync_copy(rows_v, xs_hbm.at[pl.ds(row0, HPW)])


@functools.partial(
    pl.kernel,
    out_type=jax.ShapeDtypeStruct((S, DIM), jnp.float32),
    mesh=_sc_mesh,
    scratch_types=[
        pltpu.VMEM((UCHUNK,), jnp.int32),
        pltpu.VMEM((UCHUNK, DIM), jnp.float32),
        pltpu.SemaphoreType.DMA,
    ],
    compiler_params=pltpu.CompilerParams(needs_layout_passes=False),
)
def _sc_unpermute(pos_hbm, ys_hbm, out_hbm, pid_v, rows_v, sem):
    w = lax.axis_index("s") * NC + lax.axis_index("c")
    base = w * UCHUNK
    pltpu.sync_copy(pos_hbm.at[pl.ds(base, UCHUNK)], pid_v)
    pltpu.async_copy(ys_hbm.at[pid_v], rows_v, sem).wait()
    pltpu.sync_copy(rows_v, out_hbm.at[pl.ds(base, UCHUNK)])


@jax.jit
def kernel(x, gate_w, w1, w2, balance_bias):
    b, s, d = x.shape
    xf = x.reshape(s, d)

    top1, wgt = pl.pallas_call(
        _gate_kernel,
        grid=(s // TB,),
        in_specs=[
            pl.BlockSpec((TB, DIM), lambda t: (t, 0)),
            pl.BlockSpec((E, DIM), lambda t: (0, 0)),
            pl.BlockSpec((1, E), lambda t: (0, 0)),
        ],
        out_specs=[
            pl.BlockSpec((TB, 1), lambda t: (t, 0)),
            pl.BlockSpec((TB, 1), lambda t: (t, 0)),
        ],
        out_shape=[
            jax.ShapeDtypeStruct((s, 1), jnp.int32),
            jax.ShapeDtypeStruct((s, 1), jnp.float32),
        ],
    )(xf, gate_w, balance_bias.reshape(1, E))

    x_sorted, ws, block_expert, pos, seglist = _sc_route(
        top1.reshape(s), wgt.reshape(s), xf)

    y_sorted = pl.pallas_call(
        _ffn_kernel,
        grid_spec=pltpu.PrefetchScalarGridSpec(
            num_scalar_prefetch=2,
            grid=(1,),
            in_specs=[
                pl.BlockSpec(memory_space=pl.ANY),
                pl.BlockSpec(memory_space=pl.ANY),
                pl.BlockSpec(memory_space=pl.ANY),
                pl.BlockSpec(memory_space=pl.ANY),
            ],
            out_specs=pl.BlockSpec(memory_space=pl.ANY),
            scratch_shapes=[
                pltpu.VMEM((P, DIM), jnp.float32),
                pltpu.VMEM((P, DIM), jnp.float32),
                pltpu.VMEM((P, 1), jnp.float32),
                pltpu.VMEM((3, DIM, ED), jnp.float32),
                pltpu.VMEM((3, ED, DIM), jnp.float32),
                pltpu.SemaphoreType.DMA((NCH,)),
                pltpu.SemaphoreType.DMA,
                pltpu.SemaphoreType.DMA((3, 2)),
                pltpu.SemaphoreType.DMA,
            ],
        ),
        out_shape=jax.ShapeDtypeStruct((P, DIM), jnp.float32),
        compiler_params=pltpu.CompilerParams(
            vmem_limit_bytes=100 * 1024 * 1024),
    )(block_expert, seglist, x_sorted, w1, w2, ws.reshape(P, 1))

    out = _sc_unpermute(pos, y_sorted)
    return out.reshape(b, s, d)
